# baseline (speedup 1.0000x reference)
import jax
import jax.numpy as jnp
from jax.experimental import pallas as pl


def _vq_p(x, W):
    L2 = jnp.sum(x ** 2, axis=1, keepdims=True)
    C2 = jnp.sum(W ** 2, axis=1)[None, :]
    CL = x @ W.T
    distances = L2 - 2.0 * CL + C2
    closest = jnp.argmin(distances, axis=-1)
    onehot = jnp.zeros((x.shape[0], W.shape[0]), dtype=x.dtype)
    onehot = onehot.at[jnp.arange(x.shape[0]), closest].set(1.0)
    quantized = jax.lax.dot_general(
        onehot.astype(jnp.bfloat16), W.astype(jnp.bfloat16),
        (((1,), (0,)), ((), ())),
        preferred_element_type=jnp.float32)
    return quantized, closest


def kernel(z, codebooks):
    sg = jax.lax.stop_gradient
    codebook_losses = jnp.zeros((), dtype=z.dtype)
    comitment_losses = jnp.zeros((), dtype=z.dtype)
    final_quantized = jnp.zeros_like(z)
    for i in range(codebooks.shape[0]):
        codes, code_idx = _vq_p(z, codebooks[i])
        codebook_losses = codebook_losses + jnp.mean((codes - sg(z)) ** 2)
        comitment_losses = comitment_losses + jnp.mean((sg(codes) - z) ** 2)
        codes = z + sg(codes - z)
        final_quantized = final_quantized + codes
        z = z - sg(codes)
    return final_quantized, codebook_losses, comitment_losses


# speedup vs baseline: 2.7198x; 2.7198x over previous
import functools

import jax
import jax.numpy as jnp
from jax import lax
from jax.experimental import pallas as pl
from jax.experimental.pallas import tpu as pltpu
from jax.experimental.pallas import tpu_sc as plsc

_N, _D, _K = 16384, 256, 1024
_BN = 512
_NB = _N // _BN
_NC, _NS = 2, 16
_NW = _NC * _NS
_BPW = _N // _NW
_CH = 128
_NCH = _BPW // _CH


def _rne_bf16(x):
    u = lax.bitcast_convert_type(x, jnp.int32)
    r = (u + jnp.int32(0x7FFF) + ((u >> 16) & jnp.int32(1))) & jnp.int32(-65536)
    return lax.bitcast_convert_type(r, jnp.float32)


def _chain_residual(z, qs):
    r = z
    cs = []
    for q in qs:
        qr = _rne_bf16(q)
        c = r + (qr - r)
        cs.append(c)
        r = r - c
    return r, cs


def _tc_stage_body(nprev, *refs):
    z_ref = refs[0]
    q_refs = refs[1:1 + nprev]
    w_ref = refs[1 + nprev]
    idx_ref = refs[2 + nprev]
    loss_ref = refs[3 + nprev]

    i = pl.program_id(0)
    z = z_ref[...]
    r, _ = _chain_residual(z, [q[...] for q in q_refs])
    w = w_ref[...]

    cl = lax.dot_general(
        r.astype(jnp.bfloat16), w.astype(jnp.bfloat16),
        (((1,), (1,)), ((), ())), preferred_element_type=jnp.float32)
    l2 = jnp.sum(r * r, axis=1, keepdims=True)
    c2 = jnp.sum(w * w, axis=1)[None, :]
    d = (l2 - 2.0 * cl) + c2

    m = jnp.min(d, axis=1, keepdims=True)
    cols = lax.broadcasted_iota(jnp.int32, d.shape, 1)
    idx = jnp.min(jnp.where(d == m, cols, jnp.int32(_K)), axis=1)
    idx_ref[...] = idx.reshape(1, 1, _BN)

    rows8 = lax.broadcasted_iota(jnp.int32, (8, 128), 0)
    cols8 = lax.broadcasted_iota(jnp.int32, (8, 128), 1)
    part = jnp.where((rows8 == 0) & (cols8 == 0), jnp.sum(m), 0.0)

    @pl.when(i == 0)
    def _():
        loss_ref[...] = jnp.zeros_like(loss_ref)

    loss_ref[...] += part


def _tc_stage(z, qs, w):
    nprev = len(qs)
    grid = (_NB,)
    row_spec = pl.BlockSpec((_BN, _D), lambda i: (i, 0))
    in_specs = ([row_spec] + [row_spec] * nprev
                + [pl.BlockSpec((_K, _D), lambda i: (0, 0))])
    out_specs = [
        pl.BlockSpec((1, 1, _BN), lambda i: (i, 0, 0)),
        pl.BlockSpec((8, 128), lambda i: (0, 0)),
    ]
    out_shape = [
        jax.ShapeDtypeStruct((_NB, 1, _BN), jnp.int32),
        jax.ShapeDtypeStruct((8, 128), jnp.float32),
    ]
    idx, loss = pl.pallas_call(
        functools.partial(_tc_stage_body, nprev),
        grid=grid,
        in_specs=in_specs,
        out_specs=out_specs,
        out_shape=out_shape,
    )(z, *qs, w)
    return idx.reshape(_N), loss[0, 0]


def _tc_final_body(z_ref, q1_ref, q2_ref, q3_ref, fq_ref):
    z = z_ref[...]
    _, cs = _chain_residual(z, [q1_ref[...], q2_ref[...], q3_ref[...]])
    fq_ref[...] = (cs[0] + cs[1]) + cs[2]


def _tc_final(z, q1, q2, q3):
    row_spec = pl.BlockSpec((_BN, _D), lambda i: (i, 0))
    return pl.pallas_call(
        _tc_final_body,
        grid=(_NB,),
        in_specs=[row_spec] * 4,
        out_specs=row_spec,
        out_shape=jax.ShapeDtypeStruct((_N, _D), jnp.float32),
    )(z, q1, q2, q3)


_sc_mesh = plsc.VectorSubcoreMesh(core_axis_name="c", subcore_axis_name="s")


@functools.partial(
    pl.kernel,
    out_type=jax.ShapeDtypeStruct((_N, _D), jnp.float32),
    mesh=_sc_mesh,
    scratch_types=[
        pltpu.VMEM((_NCH, _CH), jnp.int32),
        pltpu.VMEM((_CH, _D), jnp.float32),
        pltpu.SemaphoreType.DMA,
    ],
)
def _sc_gather(table_hbm, idx_hbm, out_hbm, idx_v, rows_v, sem):
    wid = lax.axis_index("s") * _NC + lax.axis_index("c")
    base = wid * _BPW
    for c in range(_NCH):
        pltpu.sync_copy(idx_hbm.at[pl.ds(base + c * _CH, _CH)], idx_v.at[c])
        pltpu.async_copy(table_hbm.at[idx_v.at[c]], rows_v, sem).wait()
        pltpu.sync_copy(rows_v, out_hbm.at[pl.ds(base + c * _CH, _CH)])


def kernel(z, codebooks):
    w1 = codebooks[0]
    w2 = codebooks[1]
    w3 = codebooks[2]

    idx1, s1 = _tc_stage(z, (), w1)
    q1 = _sc_gather(w1, idx1)
    idx2, s2 = _tc_stage(z, (q1,), w2)
    q2 = _sc_gather(w2, idx2)
    idx3, s3 = _tc_stage(z, (q1, q2), w3)
    q3 = _sc_gather(w3, idx3)
    fq = _tc_final(z, q1, q2, q3)

    total = ((s1 + s2) + s3) / jnp.float32(_N * _D)
    return fq, total, total + 0.0
